# baseline (speedup 1.0000x reference)
import functools

import jax
import jax.numpy as jnp
from jax import lax
from jax.experimental import pallas as pl
from jax.experimental.pallas import tpu as pltpu
from jax.experimental.pallas import tpu_sc as plsc

_LANES = 16


@functools.lru_cache(maxsize=None)
def _make_sc_add(B: int, S: int, D: int):
    info = plsc.get_sparse_core_info()
    NC, NS = info.num_cores, info.num_subcores
    NW = NC * NS

    rows_per_w = S // NW
    T_ROWS = 16
    n_tiles = rows_per_w // T_ROWS
    assert S % NW == 0 and rows_per_w % T_ROWS == 0 and D % _LANES == 0
    assert n_tiles >= 2

    mesh = plsc.VectorSubcoreMesh(core_axis_name="c", subcore_axis_name="s")

    @functools.partial(
        pl.kernel,
        mesh=mesh,
        out_type=jax.ShapeDtypeStruct((B * S, D), jnp.float32),
        scratch_types=(
            [pltpu.VMEM((T_ROWS, D), jnp.float32)]
            + [pltpu.VMEM((T_ROWS, D), jnp.float32)] * B
            + [pltpu.SemaphoreType.DMA] * B
            + [pltpu.SemaphoreType.DMA] * B
            + [pltpu.SemaphoreType.DMA]
        ),
    )
    def k(x_hbm, pe_hbm, out_hbm, pebuf, *rest):
        xb = rest[:B]
        ls = rest[B:2 * B]
        ss = rest[2 * B:3 * B]
        pes = rest[3 * B]
        wid = lax.axis_index("s") * NC + lax.axis_index("c")
        w_row = wid * rows_per_w

        def x_slice(t, b):
            return out_hbm.at[pl.ds(b * S + w_row + t * T_ROWS, T_ROWS)]

        def load(t, b):
            return pltpu.async_copy(
                x_hbm.at[pl.ds(b * S + w_row + t * T_ROWS, T_ROWS)],
                xb[b], ls[b])

        def add_and_store(t, b):
            xbp = xb[b]

            @plsc.parallel_loop(0, T_ROWS, unroll=1)
            def add_body(r):
                for c in range(D // _LANES):
                    sl = pl.ds(c * _LANES, _LANES)
                    plsc.addupdate(xbp.at[r, sl], pebuf[r, sl])

            return pltpu.async_copy(xbp, x_slice(t, b), ss[b])

        def pe_load(t):
            return pltpu.async_copy(
                pe_hbm.at[pl.ds(w_row + t * T_ROWS, T_ROWS)], pebuf, pes)

        h_pe = pe_load(0)
        h_loads = [load(0, b) for b in range(B)]
        h_pe.wait()
        for b in range(B):
            h_loads[b].wait()
            add_and_store(0, b)

        def tile_body(t, carry):
            for b in range(B):
                pltpu.make_async_copy(xb[b], x_slice(t, b), ss[b]).wait()
                load(t, b)
            pe_load(t).wait()
            for b in range(B):
                pltpu.make_async_copy(
                    x_hbm.at[pl.ds(b * S, T_ROWS)], xb[b], ls[b]).wait()
                add_and_store(t, b)
            return carry

        lax.fori_loop(1, n_tiles, tile_body, 0)

        for b in range(B):
            pltpu.make_async_copy(
                xb[b], x_slice(n_tiles - 1, b), ss[b]).wait()

    return k


def kernel(x, pe_emb):
    B, S, D = x.shape
    k = _make_sc_add(B, S, D)
    out = k(x.reshape(B * S, D), pe_emb)
    return out.reshape(B, S, D)
